# Initial kernel scaffold; baseline (speedup 1.0000x reference)
import jax
import jax.numpy as jnp
from jax.experimental import pallas as pl


def kernel(x, edge_index, batch, W1, b1, W2, b2, W3, b3):
    raise NotImplementedError("write your pallas kernel here")


# speedup vs baseline: 9.1490x; 9.1490x over previous
import functools

import jax
import jax.numpy as jnp
from jax import lax
from jax.experimental import pallas as pl
from jax.experimental.pallas import tpu as pltpu
from jax.experimental.pallas import tpu_sc as plsc

N_NODES = 10000
N_EDGES = 320000
D = 128
G = 64

NC = 2
NS = 16
NW = NC * NS
EPW = N_EDGES // NW
CHUNK = 80
NCHUNK = EPW // CHUNK
NPAD = 10240
RPT = NPAD // NS
ZB = RPT // 5
DEGW = 16

def _sc_scatter_body(y_hbm, src_hbm, dst_hbm, zrows_hbm, out_hbm,
                     idx_s, idx_d, rows_v, acc_sh, sem):
    cid = lax.axis_index("c")
    sid = lax.axis_index("s")
    wid = sid * NC + cid
    base = wid * EPW

    pltpu.sync_copy(zrows_hbm, rows_v)

    def zs(t, _):
        pltpu.sync_copy(rows_v, acc_sh.at[pl.ds(sid * RPT + t * CHUNK, CHUNK)])
        return 0

    lax.fori_loop(0, RPT // CHUNK, zs, 0)
    plsc.subcore_barrier()

    def body(j, _):
        pltpu.sync_copy(src_hbm.at[pl.ds(base + j * CHUNK, CHUNK)], idx_s)
        pltpu.sync_copy(dst_hbm.at[pl.ds(base + j * CHUNK, CHUNK)], idx_d)
        pltpu.async_copy(y_hbm.at[idx_s], rows_v, sem).wait()
        pltpu.sync_copy(rows_v, acc_sh.at[idx_d], add=True)
        return 0

    lax.fori_loop(0, NCHUNK, body, 0)

    plsc.subcore_barrier()

    def ro(t, _):
        r0 = sid * RPT + t * CHUNK
        pltpu.sync_copy(acc_sh.at[pl.ds(r0, CHUNK)], rows_v)
        pltpu.sync_copy(rows_v, out_hbm.at[pl.ds(cid * NPAD + r0, CHUNK)])
        return 0

    lax.fori_loop(0, RPT // CHUNK, ro, 0)


@functools.cache
def _get_sc_scatter():
    mesh = plsc.VectorSubcoreMesh(
        core_axis_name="c", subcore_axis_name="s",
        num_cores=NC, num_subcores=NS,
    )
    return pl.kernel(
        _sc_scatter_body,
        out_type=jax.ShapeDtypeStruct((NC * NPAD, D), jnp.float32),
        mesh=mesh,
        scratch_types=[
            pltpu.VMEM((CHUNK,), jnp.int32),
            pltpu.VMEM((CHUNK,), jnp.int32),
            pltpu.VMEM((CHUNK, D), jnp.float32),
            pltpu.VMEM_SHARED((NPAD, D), jnp.float32),
            pltpu.SemaphoreType.DMA,
        ],
        name="sc_edge_scatter",
    )


_RB = 2000


def _dinv_body(degt_ref, dinvb_ref):
    d = degt_ref[0, :, 0:1] + degt_ref[1, :, 0:1] + 1.0
    dinvb_ref[...] = jnp.broadcast_to(lax.rsqrt(d), (_RB, D))


def _tc_dinvb(degt):
    return pl.pallas_call(
        _dinv_body,
        grid=(N_NODES // _RB,),
        in_specs=[pl.BlockSpec((NC, _RB, D), lambda i: (0, i, 0))],
        out_specs=pl.BlockSpec((_RB, D), lambda i: (i, 0)),
        out_shape=jax.ShapeDtypeStruct((N_NODES, D), jnp.float32),
    )(degt)


def _pre1_body(x_ref, w_ref, dinv_ref, y_ref):
    y_ref[...] = (
        jnp.dot(x_ref[...], w_ref[...], preferred_element_type=jnp.float32)
        * dinv_ref[...]
    )


def _tc_pre1(x, w, dinvb):
    return pl.pallas_call(
        _pre1_body,
        grid=(N_NODES // _RB,),
        in_specs=[
            pl.BlockSpec((_RB, D), lambda i: (i, 0)),
            pl.BlockSpec((D, D), lambda i: (0, 0)),
            pl.BlockSpec((_RB, D), lambda i: (i, 0)),
        ],
        out_specs=pl.BlockSpec((_RB, D), lambda i: (i, 0)),
        out_shape=jax.ShapeDtypeStruct((N_NODES, D), jnp.float32),
    )(x, w, dinvb)


def _mid_body(s_ref, y_ref, dinv_ref, b_ref, w_ref, out_ref):
    h = (s_ref[0] + s_ref[1] + y_ref[...]) * dinv_ref[...] + b_ref[...]
    h = jnp.maximum(h, 0.0)
    out_ref[...] = (
        jnp.dot(h, w_ref[...], preferred_element_type=jnp.float32)
        * dinv_ref[...]
    )


def _tc_mid(s, y, dinvb, b, w):
    return pl.pallas_call(
        _mid_body,
        grid=(N_NODES // _RB,),
        in_specs=[
            pl.BlockSpec((NC, _RB, D), lambda i: (0, i, 0)),
            pl.BlockSpec((_RB, D), lambda i: (i, 0)),
            pl.BlockSpec((_RB, D), lambda i: (i, 0)),
            pl.BlockSpec((1, D), lambda i: (0, 0)),
            pl.BlockSpec((D, D), lambda i: (0, 0)),
        ],
        out_specs=pl.BlockSpec((_RB, D), lambda i: (i, 0)),
        out_shape=jax.ShapeDtypeStruct((N_NODES, D), jnp.float32),
    )(s, y, dinvb, b, w)


_PB = 2000


def _pool_body(s_ref, y_ref, dinv_ref, b_ref, batch_ref, out_ref, sums, counts):
    k = pl.program_id(0)

    @pl.when(k == 0)
    def _():
        sums[...] = jnp.zeros_like(sums)
        counts[...] = jnp.zeros_like(counts)

    h = (s_ref[0] + s_ref[1] + y_ref[...]) * dinv_ref[...] + b_ref[...]
    ids = batch_ref[0, 0, :]
    oh = (ids[None, :] == lax.broadcasted_iota(jnp.int32, (G, _PB), 0)).astype(
        jnp.float32
    )
    sums[...] += jnp.dot(oh, h, preferred_element_type=jnp.float32)
    counts[...] += jnp.broadcast_to(
        jnp.sum(oh, axis=1, keepdims=True), (G, D)
    )
    out_ref[...] = sums[...] / jnp.maximum(counts[...], 1.0)


def _tc_pool(s, y, dinvb, b, batch3):
    return pl.pallas_call(
        _pool_body,
        grid=(N_NODES // _PB,),
        in_specs=[
            pl.BlockSpec((NC, _PB, D), lambda k: (0, k, 0)),
            pl.BlockSpec((_PB, D), lambda k: (k, 0)),
            pl.BlockSpec((_PB, D), lambda k: (k, 0)),
            pl.BlockSpec((1, D), lambda k: (0, 0)),
            pl.BlockSpec((1, 1, _PB), lambda k: (k, 0, 0)),
        ],
        out_specs=pl.BlockSpec((G, D), lambda k: (0, 0)),
        out_shape=jax.ShapeDtypeStruct((G, D), jnp.float32),
        scratch_shapes=[
            pltpu.VMEM((G, D), jnp.float32),
            pltpu.VMEM((G, D), jnp.float32),
        ],
    )(s, y, dinvb, b, batch3)


def kernel(x, edge_index, batch, W1, b1, W2, b2, W3, b3):
    src = edge_index[0].astype(jnp.int32)
    dst = edge_index[1].astype(jnp.int32)
    batch3 = batch.astype(jnp.int32).reshape(N_NODES // _PB, 1, _PB)
    ones_t = jnp.ones((N_NODES, D), jnp.float32)
    zrows = jnp.zeros((CHUNK, D), jnp.float32)

    sc_scatter = _get_sc_scatter()

    degt = sc_scatter(ones_t, src, dst, zrows).reshape(NC, NPAD, D)[:, :N_NODES]
    dinvb = _tc_dinvb(degt)

    y1 = _tc_pre1(x, W1, dinvb)
    s1 = sc_scatter(y1, src, dst, zrows).reshape(NC, NPAD, D)[:, :N_NODES]
    y2 = _tc_mid(s1, y1, dinvb, b1.reshape(1, D), W2)
    s2 = sc_scatter(y2, src, dst, zrows).reshape(NC, NPAD, D)[:, :N_NODES]
    y3 = _tc_mid(s2, y2, dinvb, b2.reshape(1, D), W3)
    s3 = sc_scatter(y3, src, dst, zrows).reshape(NC, NPAD, D)[:, :N_NODES]
    return _tc_pool(s3, y3, dinvb, b3.reshape(1, D), batch3)


# speedup vs baseline: 20.0817x; 2.1950x over previous
import functools

import jax
import jax.numpy as jnp
from jax import lax
from jax.experimental import pallas as pl
from jax.experimental.pallas import tpu as pltpu
from jax.experimental.pallas import tpu_sc as plsc

N_NODES = 10000
N_EDGES = 320000
D = 128
G = 64

NC = 2
NS = 16
NW = NC * NS
EPW = N_EDGES // NW
CHUNK = 80
NCHUNK = EPW // CHUNK
NPAD = 10240
RPT = NPAD // NS
ZB = RPT // 5
DEGW = 16

def _sc_scatter_body(y_hbm, src_hbm, dst_hbm, zrows_hbm, out_hbm,
                     srcv, dA, dB, rowsA, rowsB, acc_sh, sem_g, sem_s):
    cid = lax.axis_index("c")
    sid = lax.axis_index("s")
    wid = sid * NC + cid
    base = wid * EPW

    pltpu.sync_copy(zrows_hbm, rowsA)

    def zs(t, _):
        pltpu.sync_copy(rowsA, acc_sh.at[pl.ds(sid * RPT + t * CHUNK, CHUNK)])
        return 0

    lax.fori_loop(0, RPT // CHUNK, zs, 0)
    plsc.subcore_barrier()

    pltpu.sync_copy(src_hbm.at[pl.ds(base, EPW)], srcv)

    def load_d(e, d):
        pltpu.sync_copy(dst_hbm.at[pl.ds(base + e * CHUNK, CHUNK)], d)

    def g_start(e, buf):
        pltpu.async_copy(y_hbm.at[srcv.at[pl.ds(e * CHUNK, CHUNK)]], buf,
                         sem_g)

    def g_wait(buf):
        pltpu.make_async_copy(
            y_hbm.at[srcv.at[pl.ds(0, CHUNK)]], buf, sem_g).wait()

    def s_start(buf, d):
        pltpu.async_copy(buf, acc_sh.at[d], sem_s, add=True)

    def s_wait(buf, d):
        pltpu.make_async_copy(buf, acc_sh.at[d], sem_s).wait()

    load_d(0, dA)
    g_start(0, rowsA)
    load_d(1, dB)
    g_start(1, rowsB)
    g_wait(rowsA)
    s_start(rowsA, dA)
    s_wait(rowsA, dA)
    load_d(2, dA)
    g_start(2, rowsA)
    g_wait(rowsB)
    s_start(rowsB, dB)

    def body(i, _):
        a = 2 * i
        s_wait(rowsB, dB)
        load_d(a + 1, dB)
        g_start(a + 1, rowsB)
        g_wait(rowsA)
        s_start(rowsA, dA)
        s_wait(rowsA, dA)
        load_d(a + 2, dA)
        g_start(a + 2, rowsA)
        g_wait(rowsB)
        s_start(rowsB, dB)
        return 0

    lax.fori_loop(1, (NCHUNK - 1) // 2, body, 0)

    s_wait(rowsB, dB)
    g_wait(rowsA)
    s_start(rowsA, dA)
    s_wait(rowsA, dA)

    plsc.subcore_barrier()

    def ro(t, _):
        r0 = sid * RPT + t * CHUNK
        pltpu.sync_copy(acc_sh.at[pl.ds(r0, CHUNK)], rowsA)
        pltpu.sync_copy(rowsA, out_hbm.at[pl.ds(cid * NPAD + r0, CHUNK)])
        return 0

    lax.fori_loop(0, RPT // CHUNK, ro, 0)


def _sc_ones_body(ones_hbm, dst_hbm, zrows_hbm, out_hbm,
                  dA, dB, rows_v, acc_sh, sem_s):
    cid = lax.axis_index("c")
    sid = lax.axis_index("s")
    wid = sid * NC + cid
    base = wid * EPW

    pltpu.sync_copy(zrows_hbm, rows_v)

    def zs(t, _):
        pltpu.sync_copy(rows_v, acc_sh.at[pl.ds(sid * RPT + t * CHUNK, CHUNK)])
        return 0

    lax.fori_loop(0, RPT // CHUNK, zs, 0)
    plsc.subcore_barrier()

    pltpu.sync_copy(ones_hbm, rows_v)

    def load_d(e, d):
        pltpu.sync_copy(dst_hbm.at[pl.ds(base + e * CHUNK, CHUNK)], d)

    def s_start(d):
        pltpu.async_copy(rows_v, acc_sh.at[d], sem_s, add=True)

    def s_wait(d):
        pltpu.make_async_copy(rows_v, acc_sh.at[d], sem_s).wait()

    load_d(0, dA)
    s_start(dA)
    load_d(1, dB)
    s_start(dB)

    def body(i, _):
        s_wait(dA)
        load_d(2 * i, dA)
        s_start(dA)
        s_wait(dB)
        load_d(2 * i + 1, dB)
        s_start(dB)
        return 0

    lax.fori_loop(1, (NCHUNK - 1) // 2, body, 0)

    s_wait(dA)
    load_d(NCHUNK - 1, dA)
    s_start(dA)
    s_wait(dB)
    s_wait(dA)

    plsc.subcore_barrier()

    def ro(t, _):
        r0 = sid * RPT + t * CHUNK
        pltpu.sync_copy(acc_sh.at[pl.ds(r0, CHUNK)], rows_v)
        pltpu.sync_copy(rows_v, out_hbm.at[pl.ds(cid * NPAD + r0, CHUNK)])
        return 0

    lax.fori_loop(0, RPT // CHUNK, ro, 0)


@functools.cache
def _get_sc_scatter():
    mesh = plsc.VectorSubcoreMesh(
        core_axis_name="c", subcore_axis_name="s",
        num_cores=NC, num_subcores=NS,
    )
    return pl.kernel(
        _sc_scatter_body,
        out_type=jax.ShapeDtypeStruct((NC * NPAD, D), jnp.float32),
        mesh=mesh,
        scratch_types=[
            pltpu.VMEM((EPW,), jnp.int32),
            pltpu.VMEM((CHUNK,), jnp.int32),
            pltpu.VMEM((CHUNK,), jnp.int32),
            pltpu.VMEM((CHUNK, D), jnp.float32),
            pltpu.VMEM((CHUNK, D), jnp.float32),
            pltpu.VMEM_SHARED((NPAD, D), jnp.float32),
            pltpu.SemaphoreType.DMA,
            pltpu.SemaphoreType.DMA,
        ],
        name="sc_edge_scatter",
    )


@functools.cache
def _get_sc_ones():
    mesh = plsc.VectorSubcoreMesh(
        core_axis_name="c", subcore_axis_name="s",
        num_cores=NC, num_subcores=NS,
    )
    return pl.kernel(
        _sc_ones_body,
        out_type=jax.ShapeDtypeStruct((NC * NPAD, D), jnp.float32),
        mesh=mesh,
        scratch_types=[
            pltpu.VMEM((CHUNK,), jnp.int32),
            pltpu.VMEM((CHUNK,), jnp.int32),
            pltpu.VMEM((CHUNK, D), jnp.float32),
            pltpu.VMEM_SHARED((NPAD, D), jnp.float32),
            pltpu.SemaphoreType.DMA,
        ],
        name="sc_degree",
    )


_RB = 2000


def _dinv_body(degt_ref, dinvb_ref):
    d = degt_ref[0, :, 0:1] + degt_ref[1, :, 0:1] + 1.0
    dinvb_ref[...] = jnp.broadcast_to(lax.rsqrt(d), (_RB, D))


def _tc_dinvb(degt):
    return pl.pallas_call(
        _dinv_body,
        grid=(N_NODES // _RB,),
        in_specs=[pl.BlockSpec((NC, _RB, D), lambda i: (0, i, 0))],
        out_specs=pl.BlockSpec((_RB, D), lambda i: (i, 0)),
        out_shape=jax.ShapeDtypeStruct((N_NODES, D), jnp.float32),
    )(degt)


def _pre1_body(x_ref, w_ref, dinv_ref, y_ref):
    y_ref[...] = (
        jnp.dot(x_ref[...], w_ref[...], preferred_element_type=jnp.float32)
        * dinv_ref[...]
    )


def _tc_pre1(x, w, dinvb):
    return pl.pallas_call(
        _pre1_body,
        grid=(N_NODES // _RB,),
        in_specs=[
            pl.BlockSpec((_RB, D), lambda i: (i, 0)),
            pl.BlockSpec((D, D), lambda i: (0, 0)),
            pl.BlockSpec((_RB, D), lambda i: (i, 0)),
        ],
        out_specs=pl.BlockSpec((_RB, D), lambda i: (i, 0)),
        out_shape=jax.ShapeDtypeStruct((N_NODES, D), jnp.float32),
    )(x, w, dinvb)


def _mid_body(s_ref, y_ref, dinv_ref, b_ref, w_ref, out_ref):
    h = (s_ref[0] + s_ref[1] + y_ref[...]) * dinv_ref[...] + b_ref[...]
    h = jnp.maximum(h, 0.0)
    out_ref[...] = (
        jnp.dot(h, w_ref[...], preferred_element_type=jnp.float32)
        * dinv_ref[...]
    )


def _tc_mid(s, y, dinvb, b, w):
    return pl.pallas_call(
        _mid_body,
        grid=(N_NODES // _RB,),
        in_specs=[
            pl.BlockSpec((NC, _RB, D), lambda i: (0, i, 0)),
            pl.BlockSpec((_RB, D), lambda i: (i, 0)),
            pl.BlockSpec((_RB, D), lambda i: (i, 0)),
            pl.BlockSpec((1, D), lambda i: (0, 0)),
            pl.BlockSpec((D, D), lambda i: (0, 0)),
        ],
        out_specs=pl.BlockSpec((_RB, D), lambda i: (i, 0)),
        out_shape=jax.ShapeDtypeStruct((N_NODES, D), jnp.float32),
    )(s, y, dinvb, b, w)


_PB = 2000


def _pool_body(s_ref, y_ref, dinv_ref, b_ref, batch_ref, out_ref, sums, counts):
    k = pl.program_id(0)

    @pl.when(k == 0)
    def _():
        sums[...] = jnp.zeros_like(sums)
        counts[...] = jnp.zeros_like(counts)

    h = (s_ref[0] + s_ref[1] + y_ref[...]) * dinv_ref[...] + b_ref[...]
    ids = batch_ref[0, 0, :]
    oh = (ids[None, :] == lax.broadcasted_iota(jnp.int32, (G, _PB), 0)).astype(
        jnp.float32
    )
    sums[...] += jnp.dot(oh, h, preferred_element_type=jnp.float32)
    counts[...] += jnp.broadcast_to(
        jnp.sum(oh, axis=1, keepdims=True), (G, D)
    )
    out_ref[...] = sums[...] / jnp.maximum(counts[...], 1.0)


def _tc_pool(s, y, dinvb, b, batch3):
    return pl.pallas_call(
        _pool_body,
        grid=(N_NODES // _PB,),
        in_specs=[
            pl.BlockSpec((NC, _PB, D), lambda k: (0, k, 0)),
            pl.BlockSpec((_PB, D), lambda k: (k, 0)),
            pl.BlockSpec((_PB, D), lambda k: (k, 0)),
            pl.BlockSpec((1, D), lambda k: (0, 0)),
            pl.BlockSpec((1, 1, _PB), lambda k: (k, 0, 0)),
        ],
        out_specs=pl.BlockSpec((G, D), lambda k: (0, 0)),
        out_shape=jax.ShapeDtypeStruct((G, D), jnp.float32),
        scratch_shapes=[
            pltpu.VMEM((G, D), jnp.float32),
            pltpu.VMEM((G, D), jnp.float32),
        ],
    )(s, y, dinvb, b, batch3)


def kernel(x, edge_index, batch, W1, b1, W2, b2, W3, b3):
    src = edge_index[0].astype(jnp.int32)
    dst = edge_index[1].astype(jnp.int32)
    batch3 = batch.astype(jnp.int32).reshape(N_NODES // _PB, 1, _PB)
    ones_r = jnp.ones((CHUNK, D), jnp.float32)
    zrows = jnp.zeros((CHUNK, D), jnp.float32)

    sc_scatter = _get_sc_scatter()
    sc_ones = _get_sc_ones()

    degt = sc_ones(ones_r, dst, zrows).reshape(NC, NPAD, D)[:, :N_NODES]
    dinvb = _tc_dinvb(degt)

    y1 = _tc_pre1(x, W1, dinvb)
    s1 = sc_scatter(y1, src, dst, zrows).reshape(NC, NPAD, D)[:, :N_NODES]
    y2 = _tc_mid(s1, y1, dinvb, b1.reshape(1, D), W2)
    s2 = sc_scatter(y2, src, dst, zrows).reshape(NC, NPAD, D)[:, :N_NODES]
    y3 = _tc_mid(s2, y2, dinvb, b2.reshape(1, D), W3)
    s3 = sc_scatter(y3, src, dst, zrows).reshape(NC, NPAD, D)[:, :N_NODES]
    return _tc_pool(s3, y3, dinvb, b3.reshape(1, D), batch3)


# speedup vs baseline: 22.5586x; 1.1233x over previous
import functools

import jax
import jax.numpy as jnp
from jax import lax
from jax.experimental import pallas as pl
from jax.experimental.pallas import tpu as pltpu
from jax.experimental.pallas import tpu_sc as plsc

N_NODES = 10000
N_EDGES = 320000
D = 128
G = 64

NC = 2
NS = 16
NW = NC * NS
EPW = N_EDGES // NW
CHUNK = 80
NCHUNK = EPW // CHUNK
NPAD = 10240
RPT = NPAD // NS
ZB = RPT // 5
DEGW = 16

def _sc_scatter_body(y_hbm, src_hbm, dst_hbm, zrows_hbm, out_hbm,
                     srcv, d0, d1, d2, rows0, rows1, rows2,
                     acc_sh, sem_g, sem_s):
    cid = lax.axis_index("c")
    sid = lax.axis_index("s")
    wid = sid * NC + cid
    base = wid * EPW

    pltpu.sync_copy(zrows_hbm, rows0)

    def zs(t, _):
        pltpu.sync_copy(rows0, acc_sh.at[pl.ds(sid * RPT + t * CHUNK, CHUNK)])
        return 0

    lax.fori_loop(0, RPT // CHUNK, zs, 0)
    plsc.subcore_barrier()

    pltpu.sync_copy(src_hbm.at[pl.ds(base, EPW)], srcv)

    def load_d(e, d):
        pltpu.sync_copy(dst_hbm.at[pl.ds(base + e * CHUNK, CHUNK)], d)

    def g_start(e, buf):
        pltpu.async_copy(y_hbm.at[srcv.at[pl.ds(e * CHUNK, CHUNK)]], buf,
                         sem_g)

    def g_wait(buf):
        pltpu.make_async_copy(
            y_hbm.at[srcv.at[pl.ds(0, CHUNK)]], buf, sem_g).wait()

    def s_start(buf, d):
        pltpu.async_copy(buf, acc_sh.at[d], sem_s, add=True)

    def s_wait(buf, d):
        pltpu.make_async_copy(buf, acc_sh.at[d], sem_s).wait()

    def step(e, p, q, do_swait=True, do_gstart=True):
        p_rows, p_d = p
        q_rows, q_d = q
        if do_swait:
            s_wait(q_rows, q_d)
        if do_gstart:
            load_d(e + 1, q_d)
            g_start(e + 1, q_rows)
        g_wait(p_rows)
        s_start(p_rows, p_d)

    s0 = (rows0, d0)
    s1 = (rows1, d1)
    s2 = (rows2, d2)

    load_d(0, d0)
    g_start(0, rows0)
    step(0, s0, s1, do_swait=False)
    step(1, s1, s2, do_swait=False)

    def body(i, _):
        e = 3 * i
        step(e + 2, s2, s0)
        step(e + 3, s0, s1)
        step(e + 4, s1, s2)
        return 0

    lax.fori_loop(0, (NCHUNK - 5) // 3, body, 0)

    step(NCHUNK - 3, s2, s0)
    step(NCHUNK - 2, s0, s1)
    step(NCHUNK - 1, s1, s2, do_gstart=False)
    s_wait(rows0, d0)
    s_wait(rows1, d1)

    plsc.subcore_barrier()

    def ro(t, _):
        r0 = sid * RPT + t * CHUNK
        pltpu.sync_copy(acc_sh.at[pl.ds(r0, CHUNK)], rows0)
        pltpu.sync_copy(rows0, out_hbm.at[pl.ds(cid * NPAD + r0, CHUNK)])
        return 0

    lax.fori_loop(0, RPT // CHUNK, ro, 0)


def _sc_ones_body(ones_hbm, dst_hbm, zrows_hbm, out_hbm,
                  dA, dB, rows_v, acc_sh, sem_s):
    cid = lax.axis_index("c")
    sid = lax.axis_index("s")
    wid = sid * NC + cid
    base = wid * EPW

    pltpu.sync_copy(zrows_hbm, rows_v)

    def zs(t, _):
        pltpu.sync_copy(rows_v, acc_sh.at[pl.ds(sid * RPT + t * CHUNK, CHUNK)])
        return 0

    lax.fori_loop(0, RPT // CHUNK, zs, 0)
    plsc.subcore_barrier()

    pltpu.sync_copy(ones_hbm, rows_v)

    def load_d(e, d):
        pltpu.sync_copy(dst_hbm.at[pl.ds(base + e * CHUNK, CHUNK)], d)

    def s_start(d):
        pltpu.async_copy(rows_v, acc_sh.at[d], sem_s, add=True)

    def s_wait(d):
        pltpu.make_async_copy(rows_v, acc_sh.at[d], sem_s).wait()

    load_d(0, dA)
    s_start(dA)
    load_d(1, dB)
    s_start(dB)

    def body(i, _):
        s_wait(dA)
        load_d(2 * i, dA)
        s_start(dA)
        s_wait(dB)
        load_d(2 * i + 1, dB)
        s_start(dB)
        return 0

    lax.fori_loop(1, (NCHUNK - 1) // 2, body, 0)

    s_wait(dA)
    load_d(NCHUNK - 1, dA)
    s_start(dA)
    s_wait(dB)
    s_wait(dA)

    plsc.subcore_barrier()

    def ro(t, _):
        r0 = sid * RPT + t * CHUNK
        pltpu.sync_copy(acc_sh.at[pl.ds(r0, CHUNK)], rows_v)
        pltpu.sync_copy(rows_v, out_hbm.at[pl.ds(cid * NPAD + r0, CHUNK)])
        return 0

    lax.fori_loop(0, RPT // CHUNK, ro, 0)


@functools.cache
def _get_sc_scatter():
    mesh = plsc.VectorSubcoreMesh(
        core_axis_name="c", subcore_axis_name="s",
        num_cores=NC, num_subcores=NS,
    )
    return pl.kernel(
        _sc_scatter_body,
        out_type=jax.ShapeDtypeStruct((NC * NPAD, D), jnp.float32),
        mesh=mesh,
        scratch_types=[
            pltpu.VMEM((EPW,), jnp.int32),
            pltpu.VMEM((CHUNK,), jnp.int32),
            pltpu.VMEM((CHUNK,), jnp.int32),
            pltpu.VMEM((CHUNK,), jnp.int32),
            pltpu.VMEM((CHUNK, D), jnp.float32),
            pltpu.VMEM((CHUNK, D), jnp.float32),
            pltpu.VMEM((CHUNK, D), jnp.float32),
            pltpu.VMEM_SHARED((NPAD, D), jnp.float32),
            pltpu.SemaphoreType.DMA,
            pltpu.SemaphoreType.DMA,
        ],
        name="sc_edge_scatter",
    )


@functools.cache
def _get_sc_ones():
    mesh = plsc.VectorSubcoreMesh(
        core_axis_name="c", subcore_axis_name="s",
        num_cores=NC, num_subcores=NS,
    )
    return pl.kernel(
        _sc_ones_body,
        out_type=jax.ShapeDtypeStruct((NC * NPAD, D), jnp.float32),
        mesh=mesh,
        scratch_types=[
            pltpu.VMEM((CHUNK,), jnp.int32),
            pltpu.VMEM((CHUNK,), jnp.int32),
            pltpu.VMEM((CHUNK, D), jnp.float32),
            pltpu.VMEM_SHARED((NPAD, D), jnp.float32),
            pltpu.SemaphoreType.DMA,
        ],
        name="sc_degree",
    )


_RB = 2000


def _dinv_body(degt_ref, dinvb_ref):
    d = degt_ref[0, :, 0:1] + degt_ref[1, :, 0:1] + 1.0
    dinvb_ref[...] = jnp.broadcast_to(lax.rsqrt(d), (_RB, D))


def _tc_dinvb(degt):
    return pl.pallas_call(
        _dinv_body,
        grid=(N_NODES // _RB,),
        in_specs=[pl.BlockSpec((NC, _RB, D), lambda i: (0, i, 0))],
        out_specs=pl.BlockSpec((_RB, D), lambda i: (i, 0)),
        out_shape=jax.ShapeDtypeStruct((N_NODES, D), jnp.float32),
    )(degt)


def _pre1_body(x_ref, w_ref, dinv_ref, y_ref):
    y_ref[...] = (
        jnp.dot(x_ref[...], w_ref[...], preferred_element_type=jnp.float32)
        * dinv_ref[...]
    )


def _tc_pre1(x, w, dinvb):
    return pl.pallas_call(
        _pre1_body,
        grid=(N_NODES // _RB,),
        in_specs=[
            pl.BlockSpec((_RB, D), lambda i: (i, 0)),
            pl.BlockSpec((D, D), lambda i: (0, 0)),
            pl.BlockSpec((_RB, D), lambda i: (i, 0)),
        ],
        out_specs=pl.BlockSpec((_RB, D), lambda i: (i, 0)),
        out_shape=jax.ShapeDtypeStruct((N_NODES, D), jnp.float32),
    )(x, w, dinvb)


def _mid_body(s_ref, y_ref, dinv_ref, b_ref, w_ref, out_ref):
    h = (s_ref[0] + s_ref[1] + y_ref[...]) * dinv_ref[...] + b_ref[...]
    h = jnp.maximum(h, 0.0)
    out_ref[...] = (
        jnp.dot(h, w_ref[...], preferred_element_type=jnp.float32)
        * dinv_ref[...]
    )


def _tc_mid(s, y, dinvb, b, w):
    return pl.pallas_call(
        _mid_body,
        grid=(N_NODES // _RB,),
        in_specs=[
            pl.BlockSpec((NC, _RB, D), lambda i: (0, i, 0)),
            pl.BlockSpec((_RB, D), lambda i: (i, 0)),
            pl.BlockSpec((_RB, D), lambda i: (i, 0)),
            pl.BlockSpec((1, D), lambda i: (0, 0)),
            pl.BlockSpec((D, D), lambda i: (0, 0)),
        ],
        out_specs=pl.BlockSpec((_RB, D), lambda i: (i, 0)),
        out_shape=jax.ShapeDtypeStruct((N_NODES, D), jnp.float32),
    )(s, y, dinvb, b, w)


_PB = 2000


def _pool_body(s_ref, y_ref, dinv_ref, b_ref, batch_ref, out_ref, sums, counts):
    k = pl.program_id(0)

    @pl.when(k == 0)
    def _():
        sums[...] = jnp.zeros_like(sums)
        counts[...] = jnp.zeros_like(counts)

    h = (s_ref[0] + s_ref[1] + y_ref[...]) * dinv_ref[...] + b_ref[...]
    ids = batch_ref[0, 0, :]
    oh = (ids[None, :] == lax.broadcasted_iota(jnp.int32, (G, _PB), 0)).astype(
        jnp.float32
    )
    sums[...] += jnp.dot(oh, h, preferred_element_type=jnp.float32)
    counts[...] += jnp.broadcast_to(
        jnp.sum(oh, axis=1, keepdims=True), (G, D)
    )
    out_ref[...] = sums[...] / jnp.maximum(counts[...], 1.0)


def _tc_pool(s, y, dinvb, b, batch3):
    return pl.pallas_call(
        _pool_body,
        grid=(N_NODES // _PB,),
        in_specs=[
            pl.BlockSpec((NC, _PB, D), lambda k: (0, k, 0)),
            pl.BlockSpec((_PB, D), lambda k: (k, 0)),
            pl.BlockSpec((_PB, D), lambda k: (k, 0)),
            pl.BlockSpec((1, D), lambda k: (0, 0)),
            pl.BlockSpec((1, 1, _PB), lambda k: (k, 0, 0)),
        ],
        out_specs=pl.BlockSpec((G, D), lambda k: (0, 0)),
        out_shape=jax.ShapeDtypeStruct((G, D), jnp.float32),
        scratch_shapes=[
            pltpu.VMEM((G, D), jnp.float32),
            pltpu.VMEM((G, D), jnp.float32),
        ],
    )(s, y, dinvb, b, batch3)


def kernel(x, edge_index, batch, W1, b1, W2, b2, W3, b3):
    src = edge_index[0].astype(jnp.int32)
    dst = edge_index[1].astype(jnp.int32)
    batch3 = batch.astype(jnp.int32).reshape(N_NODES // _PB, 1, _PB)
    ones_r = jnp.ones((CHUNK, D), jnp.float32)
    zrows = jnp.zeros((CHUNK, D), jnp.float32)

    sc_scatter = _get_sc_scatter()
    sc_ones = _get_sc_ones()

    degt = sc_ones(ones_r, dst, zrows).reshape(NC, NPAD, D)[:, :N_NODES]
    dinvb = _tc_dinvb(degt)

    y1 = _tc_pre1(x, W1, dinvb)
    s1 = sc_scatter(y1, src, dst, zrows).reshape(NC, NPAD, D)[:, :N_NODES]
    y2 = _tc_mid(s1, y1, dinvb, b1.reshape(1, D), W2)
    s2 = sc_scatter(y2, src, dst, zrows).reshape(NC, NPAD, D)[:, :N_NODES]
    y3 = _tc_mid(s2, y2, dinvb, b2.reshape(1, D), W3)
    s3 = sc_scatter(y3, src, dst, zrows).reshape(NC, NPAD, D)[:, :N_NODES]
    return _tc_pool(s3, y3, dinvb, b3.reshape(1, D), batch3)


# speedup vs baseline: 22.7852x; 1.0100x over previous
import functools

import jax
import jax.numpy as jnp
from jax import lax
from jax.experimental import pallas as pl
from jax.experimental.pallas import tpu as pltpu
from jax.experimental.pallas import tpu_sc as plsc

N_NODES = 10000
N_EDGES = 320000
D = 128
G = 64

NC = 2
NS = 16
NW = NC * NS
EPW = N_EDGES // NW
CHUNK = 80
NCHUNK = EPW // CHUNK
NPAD = 10240
RPT = NPAD // NS
ZB = RPT // 5
DEGW = 16

def _sc_scatter_body(y_hbm, src_hbm, dst_hbm, zrows_hbm, out_hbm,
                     srcv, d0, d1, d2, rows0, rows1, rows2,
                     acc_sh, sem_g, sem_s):
    cid = lax.axis_index("c")
    sid = lax.axis_index("s")
    wid = sid * NC + cid
    base = wid * EPW

    pltpu.sync_copy(zrows_hbm, rows0)

    def zs(t, _):
        pltpu.sync_copy(rows0, acc_sh.at[pl.ds(sid * RPT + t * CHUNK, CHUNK)])
        return 0

    lax.fori_loop(0, RPT // CHUNK, zs, 0)
    plsc.subcore_barrier()

    pltpu.sync_copy(src_hbm.at[pl.ds(base, EPW)], srcv)

    def load_d(e, d):
        pltpu.sync_copy(dst_hbm.at[pl.ds(base + e * CHUNK, CHUNK)], d)

    def g_start(e, buf):
        pltpu.async_copy(y_hbm.at[srcv.at[pl.ds(e * CHUNK, CHUNK)]], buf,
                         sem_g)

    def g_wait(buf):
        pltpu.make_async_copy(
            y_hbm.at[srcv.at[pl.ds(0, CHUNK)]], buf, sem_g).wait()

    def s_start(buf, d):
        pltpu.async_copy(buf, acc_sh.at[d], sem_s, add=True)

    def s_wait(buf, d):
        pltpu.make_async_copy(buf, acc_sh.at[d], sem_s).wait()

    def step(e, p, q, do_swait=True, do_gstart=True):
        p_rows, p_d = p
        q_rows, q_d = q
        if do_swait:
            s_wait(q_rows, q_d)
        if do_gstart:
            load_d(e + 1, q_d)
            g_start(e + 1, q_rows)
        g_wait(p_rows)
        s_start(p_rows, p_d)

    s0 = (rows0, d0)
    s1 = (rows1, d1)
    s2 = (rows2, d2)

    load_d(0, d0)
    g_start(0, rows0)
    step(0, s0, s1, do_swait=False)
    step(1, s1, s2, do_swait=False)

    def body(i, _):
        e = 3 * i
        step(e + 2, s2, s0)
        step(e + 3, s0, s1)
        step(e + 4, s1, s2)
        return 0

    lax.fori_loop(0, (NCHUNK - 5) // 3, body, 0)

    step(NCHUNK - 3, s2, s0)
    step(NCHUNK - 2, s0, s1)
    step(NCHUNK - 1, s1, s2, do_gstart=False)
    s_wait(rows0, d0)
    s_wait(rows1, d1)

    plsc.subcore_barrier()

    def ro(t, _):
        r0 = sid * RPT + t * CHUNK
        pltpu.sync_copy(acc_sh.at[pl.ds(r0, CHUNK)], rows0)
        pltpu.sync_copy(rows0, out_hbm.at[pl.ds(cid * NPAD + r0, CHUNK)])
        return 0

    lax.fori_loop(0, RPT // CHUNK, ro, 0)


def _sc_ones_body(ones_hbm, dst_hbm, zrows_hbm, out_hbm,
                  dA, dB, rows_v, acc_sh, sem_s):
    cid = lax.axis_index("c")
    sid = lax.axis_index("s")
    wid = sid * NC + cid
    base = wid * EPW

    pltpu.sync_copy(zrows_hbm, rows_v)

    def zs(t, _):
        pltpu.sync_copy(rows_v, acc_sh.at[pl.ds(sid * RPT + t * CHUNK, CHUNK)])
        return 0

    lax.fori_loop(0, RPT // CHUNK, zs, 0)
    plsc.subcore_barrier()

    pltpu.sync_copy(ones_hbm, rows_v)

    def load_d(e, d):
        pltpu.sync_copy(dst_hbm.at[pl.ds(base + e * CHUNK, CHUNK)], d)

    def s_start(d):
        pltpu.async_copy(rows_v, acc_sh.at[d], sem_s, add=True)

    def s_wait(d):
        pltpu.make_async_copy(rows_v, acc_sh.at[d], sem_s).wait()

    load_d(0, dA)
    s_start(dA)
    load_d(1, dB)
    s_start(dB)

    def body(i, _):
        s_wait(dA)
        load_d(2 * i, dA)
        s_start(dA)
        s_wait(dB)
        load_d(2 * i + 1, dB)
        s_start(dB)
        return 0

    lax.fori_loop(1, (NCHUNK - 1) // 2, body, 0)

    s_wait(dA)
    load_d(NCHUNK - 1, dA)
    s_start(dA)
    s_wait(dB)
    s_wait(dA)

    plsc.subcore_barrier()

    def ro(t, _):
        r0 = sid * RPT + t * CHUNK
        pltpu.sync_copy(acc_sh.at[pl.ds(r0, CHUNK)], rows_v)
        pltpu.sync_copy(rows_v, out_hbm.at[pl.ds(cid * NPAD + r0, CHUNK)])
        return 0

    lax.fori_loop(0, RPT // CHUNK, ro, 0)


@functools.cache
def _get_sc_scatter():
    mesh = plsc.VectorSubcoreMesh(
        core_axis_name="c", subcore_axis_name="s",
        num_cores=NC, num_subcores=NS,
    )
    return pl.kernel(
        _sc_scatter_body,
        out_type=jax.ShapeDtypeStruct((NC * NPAD, D), jnp.float32),
        mesh=mesh,
        scratch_types=[
            pltpu.VMEM((EPW,), jnp.int32),
            pltpu.VMEM((CHUNK,), jnp.int32),
            pltpu.VMEM((CHUNK,), jnp.int32),
            pltpu.VMEM((CHUNK,), jnp.int32),
            pltpu.VMEM((CHUNK, D), jnp.float32),
            pltpu.VMEM((CHUNK, D), jnp.float32),
            pltpu.VMEM((CHUNK, D), jnp.float32),
            pltpu.VMEM_SHARED((NPAD, D), jnp.float32),
            pltpu.SemaphoreType.DMA,
            pltpu.SemaphoreType.DMA,
        ],
        name="sc_edge_scatter",
    )


@functools.cache
def _get_sc_ones():
    mesh = plsc.VectorSubcoreMesh(
        core_axis_name="c", subcore_axis_name="s",
        num_cores=NC, num_subcores=NS,
    )
    return pl.kernel(
        _sc_ones_body,
        out_type=jax.ShapeDtypeStruct((NC * NPAD, D), jnp.float32),
        mesh=mesh,
        scratch_types=[
            pltpu.VMEM((CHUNK,), jnp.int32),
            pltpu.VMEM((CHUNK,), jnp.int32),
            pltpu.VMEM((CHUNK, D), jnp.float32),
            pltpu.VMEM_SHARED((NPAD, D), jnp.float32),
            pltpu.SemaphoreType.DMA,
        ],
        name="sc_degree",
    )


_RB = 2000


def _xw_body(x_ref, w_ref, y_ref):
    y_ref[...] = jnp.dot(
        x_ref[...], w_ref[...], preferred_element_type=jnp.float32
    )


def _tc_xw(x, w):
    return pl.pallas_call(
        _xw_body,
        grid=(N_NODES // _RB,),
        in_specs=[
            pl.BlockSpec((_RB, D), lambda i: (i, 0)),
            pl.BlockSpec((D, D), lambda i: (0, 0)),
        ],
        out_specs=pl.BlockSpec((_RB, D), lambda i: (i, 0)),
        out_shape=jax.ShapeDtypeStruct((N_NODES, D), jnp.float32),
    )(x, w)


def _scale_body(degt_ref, xw_ref, dinvb_ref, y_ref):
    d = degt_ref[0, :, 0:1] + degt_ref[1, :, 0:1] + 1.0
    dinvb = jnp.broadcast_to(lax.rsqrt(d), (_RB, D))
    dinvb_ref[...] = dinvb
    y_ref[...] = xw_ref[...] * dinvb


def _tc_scale(degt, xw):
    return pl.pallas_call(
        _scale_body,
        grid=(N_NODES // _RB,),
        in_specs=[
            pl.BlockSpec((NC, _RB, D), lambda i: (0, i, 0)),
            pl.BlockSpec((_RB, D), lambda i: (i, 0)),
        ],
        out_specs=[
            pl.BlockSpec((_RB, D), lambda i: (i, 0)),
            pl.BlockSpec((_RB, D), lambda i: (i, 0)),
        ],
        out_shape=[
            jax.ShapeDtypeStruct((N_NODES, D), jnp.float32),
            jax.ShapeDtypeStruct((N_NODES, D), jnp.float32),
        ],
    )(degt, xw)


def _mid_body(s_ref, y_ref, dinv_ref, b_ref, w_ref, out_ref):
    h = (s_ref[0] + s_ref[1] + y_ref[...]) * dinv_ref[...] + b_ref[...]
    h = jnp.maximum(h, 0.0)
    out_ref[...] = (
        jnp.dot(h, w_ref[...], preferred_element_type=jnp.float32)
        * dinv_ref[...]
    )


def _tc_mid(s, y, dinvb, b, w):
    return pl.pallas_call(
        _mid_body,
        grid=(N_NODES // _RB,),
        in_specs=[
            pl.BlockSpec((NC, _RB, D), lambda i: (0, i, 0)),
            pl.BlockSpec((_RB, D), lambda i: (i, 0)),
            pl.BlockSpec((_RB, D), lambda i: (i, 0)),
            pl.BlockSpec((1, D), lambda i: (0, 0)),
            pl.BlockSpec((D, D), lambda i: (0, 0)),
        ],
        out_specs=pl.BlockSpec((_RB, D), lambda i: (i, 0)),
        out_shape=jax.ShapeDtypeStruct((N_NODES, D), jnp.float32),
    )(s, y, dinvb, b, w)


_PB = 2000


def _pool_body(s_ref, y_ref, dinv_ref, b_ref, batch_ref, out_ref, sums, counts):
    k = pl.program_id(0)

    @pl.when(k == 0)
    def _():
        sums[...] = jnp.zeros_like(sums)
        counts[...] = jnp.zeros_like(counts)

    h = (s_ref[0] + s_ref[1] + y_ref[...]) * dinv_ref[...] + b_ref[...]
    ids = batch_ref[0, 0, :]
    oh = (ids[None, :] == lax.broadcasted_iota(jnp.int32, (G, _PB), 0)).astype(
        jnp.float32
    )
    sums[...] += jnp.dot(oh, h, preferred_element_type=jnp.float32)
    counts[...] += jnp.broadcast_to(
        jnp.sum(oh, axis=1, keepdims=True), (G, D)
    )
    out_ref[...] = sums[...] / jnp.maximum(counts[...], 1.0)


def _tc_pool(s, y, dinvb, b, batch3):
    return pl.pallas_call(
        _pool_body,
        grid=(N_NODES // _PB,),
        in_specs=[
            pl.BlockSpec((NC, _PB, D), lambda k: (0, k, 0)),
            pl.BlockSpec((_PB, D), lambda k: (k, 0)),
            pl.BlockSpec((_PB, D), lambda k: (k, 0)),
            pl.BlockSpec((1, D), lambda k: (0, 0)),
            pl.BlockSpec((1, 1, _PB), lambda k: (k, 0, 0)),
        ],
        out_specs=pl.BlockSpec((G, D), lambda k: (0, 0)),
        out_shape=jax.ShapeDtypeStruct((G, D), jnp.float32),
        scratch_shapes=[
            pltpu.VMEM((G, D), jnp.float32),
            pltpu.VMEM((G, D), jnp.float32),
        ],
    )(s, y, dinvb, b, batch3)


def kernel(x, edge_index, batch, W1, b1, W2, b2, W3, b3):
    src = edge_index[0].astype(jnp.int32)
    dst = edge_index[1].astype(jnp.int32)
    batch3 = batch.astype(jnp.int32).reshape(N_NODES // _PB, 1, _PB)
    ones_r = jnp.ones((CHUNK, D), jnp.float32)
    zrows = jnp.zeros((CHUNK, D), jnp.float32)

    sc_scatter = _get_sc_scatter()
    sc_ones = _get_sc_ones()

    xw1 = _tc_xw(x, W1)
    degt = sc_ones(ones_r, dst, zrows).reshape(NC, NPAD, D)[:, :N_NODES]
    dinvb, y1 = _tc_scale(degt, xw1)
    s1 = sc_scatter(y1, src, dst, zrows).reshape(NC, NPAD, D)[:, :N_NODES]
    y2 = _tc_mid(s1, y1, dinvb, b1.reshape(1, D), W2)
    s2 = sc_scatter(y2, src, dst, zrows).reshape(NC, NPAD, D)[:, :N_NODES]
    y3 = _tc_mid(s2, y2, dinvb, b2.reshape(1, D), W3)
    s3 = sc_scatter(y3, src, dst, zrows).reshape(NC, NPAD, D)[:, :N_NODES]
    return _tc_pool(s3, y3, dinvb, b3.reshape(1, D), batch3)


# speedup vs baseline: 22.9514x; 1.0073x over previous
import functools

import jax
import jax.numpy as jnp
from jax import lax
from jax.experimental import pallas as pl
from jax.experimental.pallas import tpu as pltpu
from jax.experimental.pallas import tpu_sc as plsc

N_NODES = 10000
N_EDGES = 320000
D = 128
G = 64

NC = 2
NS = 16
NW = NC * NS
EPW = N_EDGES // NW
CHUNK = 80
NCHUNK = EPW // CHUNK
NPAD = 10240
RPT = NPAD // NS
ZB = RPT // 5
DEGW = 16

def _sc_scatter_body(y_hbm, src_hbm, dst_hbm, zrows_hbm, out_hbm,
                     srcv, d0, d1, d2, rows0, rows1, rows2,
                     acc_sh, sem_g, sem_s):
    cid = lax.axis_index("c")
    sid = lax.axis_index("s")
    wid = sid * NC + cid
    base = wid * EPW

    pltpu.sync_copy(zrows_hbm, rows0)

    def zs(t, _):
        pltpu.sync_copy(rows0, acc_sh.at[pl.ds(sid * RPT + t * CHUNK, CHUNK)])
        return 0

    lax.fori_loop(0, RPT // CHUNK, zs, 0)
    plsc.subcore_barrier()

    pltpu.sync_copy(src_hbm.at[pl.ds(base, EPW)], srcv)

    def load_d(e, d):
        pltpu.sync_copy(dst_hbm.at[pl.ds(base + e * CHUNK, CHUNK)], d)

    def g_start(e, buf):
        pltpu.async_copy(y_hbm.at[srcv.at[pl.ds(e * CHUNK, CHUNK)]], buf,
                         sem_g)

    def g_wait(buf):
        pltpu.make_async_copy(
            y_hbm.at[srcv.at[pl.ds(0, CHUNK)]], buf, sem_g).wait()

    def s_start(buf, d):
        pltpu.async_copy(buf, acc_sh.at[d], sem_s, add=True)

    def s_wait(buf, d):
        pltpu.make_async_copy(buf, acc_sh.at[d], sem_s).wait()

    def step(e, p, q, do_swait=True, do_gstart=True):
        p_rows, p_d = p
        q_rows, q_d = q
        if do_swait:
            s_wait(q_rows, q_d)
        if do_gstart:
            load_d(e + 1, q_d)
            g_start(e + 1, q_rows)
        g_wait(p_rows)
        s_start(p_rows, p_d)

    s0 = (rows0, d0)
    s1 = (rows1, d1)
    s2 = (rows2, d2)

    load_d(0, d0)
    g_start(0, rows0)
    step(0, s0, s1, do_swait=False)
    step(1, s1, s2, do_swait=False)

    def body(i, _):
        e = 3 * i
        step(e + 2, s2, s0)
        step(e + 3, s0, s1)
        step(e + 4, s1, s2)
        return 0

    lax.fori_loop(0, (NCHUNK - 5) // 3, body, 0)

    step(NCHUNK - 3, s2, s0)
    step(NCHUNK - 2, s0, s1)
    step(NCHUNK - 1, s1, s2, do_gstart=False)
    s_wait(rows0, d0)
    s_wait(rows1, d1)

    plsc.subcore_barrier()

    def ro(t, _):
        r0 = sid * RPT + t * CHUNK
        pltpu.sync_copy(acc_sh.at[pl.ds(r0, CHUNK)], rows0)
        pltpu.sync_copy(rows0, out_hbm.at[pl.ds(cid * NPAD + r0, CHUNK)])
        return 0

    lax.fori_loop(0, RPT // CHUNK, ro, 0)


def _sc_ones_body(ones_hbm, dst_hbm, zdeg_hbm, out_hbm,
                  dA, dB, ones_v, zb_v, acc_sh, sem_s):
    cid = lax.axis_index("c")
    sid = lax.axis_index("s")
    wid = sid * NC + cid
    base = wid * EPW

    pltpu.sync_copy(zdeg_hbm, zb_v)
    pltpu.sync_copy(zb_v, acc_sh.at[pl.ds(sid * RPT, RPT)])
    plsc.subcore_barrier()

    pltpu.sync_copy(ones_hbm, ones_v)

    def load_d(e, d):
        pltpu.sync_copy(dst_hbm.at[pl.ds(base + e * CHUNK, CHUNK)], d)

    def s_start(d):
        pltpu.async_copy(ones_v, acc_sh.at[d], sem_s, add=True)

    def s_wait(d):
        pltpu.make_async_copy(ones_v, acc_sh.at[d], sem_s).wait()

    load_d(0, dA)
    s_start(dA)
    load_d(1, dB)
    s_start(dB)

    def body(i, _):
        s_wait(dA)
        load_d(2 * i, dA)
        s_start(dA)
        s_wait(dB)
        load_d(2 * i + 1, dB)
        s_start(dB)
        return 0

    lax.fori_loop(1, (NCHUNK - 1) // 2, body, 0)

    s_wait(dA)
    load_d(NCHUNK - 1, dA)
    s_start(dA)
    s_wait(dB)
    s_wait(dA)

    plsc.subcore_barrier()
    pltpu.sync_copy(acc_sh.at[pl.ds(sid * RPT, RPT)], zb_v)
    pltpu.sync_copy(zb_v, out_hbm.at[pl.ds(cid * NPAD + sid * RPT, RPT)])


@functools.cache
def _get_sc_scatter():
    mesh = plsc.VectorSubcoreMesh(
        core_axis_name="c", subcore_axis_name="s",
        num_cores=NC, num_subcores=NS,
    )
    return pl.kernel(
        _sc_scatter_body,
        out_type=jax.ShapeDtypeStruct((NC * NPAD, D), jnp.float32),
        mesh=mesh,
        scratch_types=[
            pltpu.VMEM((EPW,), jnp.int32),
            pltpu.VMEM((CHUNK,), jnp.int32),
            pltpu.VMEM((CHUNK,), jnp.int32),
            pltpu.VMEM((CHUNK,), jnp.int32),
            pltpu.VMEM((CHUNK, D), jnp.float32),
            pltpu.VMEM((CHUNK, D), jnp.float32),
            pltpu.VMEM((CHUNK, D), jnp.float32),
            pltpu.VMEM_SHARED((NPAD, D), jnp.float32),
            pltpu.SemaphoreType.DMA,
            pltpu.SemaphoreType.DMA,
        ],
        name="sc_edge_scatter",
    )


@functools.cache
def _get_sc_ones():
    mesh = plsc.VectorSubcoreMesh(
        core_axis_name="c", subcore_axis_name="s",
        num_cores=NC, num_subcores=NS,
    )
    return pl.kernel(
        _sc_ones_body,
        out_type=jax.ShapeDtypeStruct((NC * NPAD, DEGW), jnp.float32),
        mesh=mesh,
        scratch_types=[
            pltpu.VMEM((CHUNK,), jnp.int32),
            pltpu.VMEM((CHUNK,), jnp.int32),
            pltpu.VMEM((CHUNK, DEGW), jnp.float32),
            pltpu.VMEM((RPT, DEGW), jnp.float32),
            pltpu.VMEM_SHARED((NPAD, DEGW), jnp.float32),
            pltpu.SemaphoreType.DMA,
        ],
        compiler_params=pltpu.CompilerParams(use_tc_tiling_on_sc=False),
        name="sc_degree",
    )


_RB = 2000


def _xw_body(x_ref, w_ref, y_ref):
    y_ref[...] = jnp.dot(
        x_ref[...], w_ref[...], preferred_element_type=jnp.float32
    )


def _tc_xw(x, w):
    return pl.pallas_call(
        _xw_body,
        grid=(N_NODES // _RB,),
        in_specs=[
            pl.BlockSpec((_RB, D), lambda i: (i, 0)),
            pl.BlockSpec((D, D), lambda i: (0, 0)),
        ],
        out_specs=pl.BlockSpec((_RB, D), lambda i: (i, 0)),
        out_shape=jax.ShapeDtypeStruct((N_NODES, D), jnp.float32),
    )(x, w)


def _scale_body(degt_ref, xw_ref, dinvb_ref, y_ref):
    d = degt_ref[0, :, 0:1] + degt_ref[1, :, 0:1] + 1.0
    dinvb = jnp.broadcast_to(lax.rsqrt(d), (_RB, D))
    dinvb_ref[...] = dinvb
    y_ref[...] = xw_ref[...] * dinvb


def _tc_scale(degt, xw):
    return pl.pallas_call(
        _scale_body,
        grid=(N_NODES // _RB,),
        in_specs=[
            pl.BlockSpec((NC, _RB, DEGW), lambda i: (0, i, 0)),
            pl.BlockSpec((_RB, D), lambda i: (i, 0)),
        ],
        out_specs=[
            pl.BlockSpec((_RB, D), lambda i: (i, 0)),
            pl.BlockSpec((_RB, D), lambda i: (i, 0)),
        ],
        out_shape=[
            jax.ShapeDtypeStruct((N_NODES, D), jnp.float32),
            jax.ShapeDtypeStruct((N_NODES, D), jnp.float32),
        ],
    )(degt, xw)


def _mid_body(s_ref, y_ref, dinv_ref, b_ref, w_ref, out_ref):
    h = (s_ref[0] + s_ref[1] + y_ref[...]) * dinv_ref[...] + b_ref[...]
    h = jnp.maximum(h, 0.0)
    out_ref[...] = (
        jnp.dot(h, w_ref[...], preferred_element_type=jnp.float32)
        * dinv_ref[...]
    )


def _tc_mid(s, y, dinvb, b, w):
    return pl.pallas_call(
        _mid_body,
        grid=(N_NODES // _RB,),
        in_specs=[
            pl.BlockSpec((NC, _RB, D), lambda i: (0, i, 0)),
            pl.BlockSpec((_RB, D), lambda i: (i, 0)),
            pl.BlockSpec((_RB, D), lambda i: (i, 0)),
            pl.BlockSpec((1, D), lambda i: (0, 0)),
            pl.BlockSpec((D, D), lambda i: (0, 0)),
        ],
        out_specs=pl.BlockSpec((_RB, D), lambda i: (i, 0)),
        out_shape=jax.ShapeDtypeStruct((N_NODES, D), jnp.float32),
    )(s, y, dinvb, b, w)


_PB = 2000


def _pool_body(s_ref, y_ref, dinv_ref, b_ref, batch_ref, out_ref, sums, counts):
    k = pl.program_id(0)

    @pl.when(k == 0)
    def _():
        sums[...] = jnp.zeros_like(sums)
        counts[...] = jnp.zeros_like(counts)

    h = (s_ref[0] + s_ref[1] + y_ref[...]) * dinv_ref[...] + b_ref[...]
    ids = batch_ref[0, 0, :]
    oh = (ids[None, :] == lax.broadcasted_iota(jnp.int32, (G, _PB), 0)).astype(
        jnp.float32
    )
    sums[...] += jnp.dot(oh, h, preferred_element_type=jnp.float32)
    counts[...] += jnp.broadcast_to(
        jnp.sum(oh, axis=1, keepdims=True), (G, D)
    )
    out_ref[...] = sums[...] / jnp.maximum(counts[...], 1.0)


def _tc_pool(s, y, dinvb, b, batch3):
    return pl.pallas_call(
        _pool_body,
        grid=(N_NODES // _PB,),
        in_specs=[
            pl.BlockSpec((NC, _PB, D), lambda k: (0, k, 0)),
            pl.BlockSpec((_PB, D), lambda k: (k, 0)),
            pl.BlockSpec((_PB, D), lambda k: (k, 0)),
            pl.BlockSpec((1, D), lambda k: (0, 0)),
            pl.BlockSpec((1, 1, _PB), lambda k: (k, 0, 0)),
        ],
        out_specs=pl.BlockSpec((G, D), lambda k: (0, 0)),
        out_shape=jax.ShapeDtypeStruct((G, D), jnp.float32),
        scratch_shapes=[
            pltpu.VMEM((G, D), jnp.float32),
            pltpu.VMEM((G, D), jnp.float32),
        ],
    )(s, y, dinvb, b, batch3)


def kernel(x, edge_index, batch, W1, b1, W2, b2, W3, b3):
    src = edge_index[0].astype(jnp.int32)
    dst = edge_index[1].astype(jnp.int32)
    batch3 = batch.astype(jnp.int32).reshape(N_NODES // _PB, 1, _PB)
    ones_r = jnp.ones((CHUNK, DEGW), jnp.float32)
    zdeg = jnp.zeros((RPT, DEGW), jnp.float32)
    zrows = jnp.zeros((CHUNK, D), jnp.float32)

    sc_scatter = _get_sc_scatter()
    sc_ones = _get_sc_ones()

    xw1 = _tc_xw(x, W1)
    degt = sc_ones(ones_r, dst, zdeg).reshape(NC, NPAD, DEGW)[:, :N_NODES]
    dinvb, y1 = _tc_scale(degt, xw1)
    s1 = sc_scatter(y1, src, dst, zrows).reshape(NC, NPAD, D)[:, :N_NODES]
    y2 = _tc_mid(s1, y1, dinvb, b1.reshape(1, D), W2)
    s2 = sc_scatter(y2, src, dst, zrows).reshape(NC, NPAD, D)[:, :N_NODES]
    y3 = _tc_mid(s2, y2, dinvb, b2.reshape(1, D), W3)
    s3 = sc_scatter(y3, src, dst, zrows).reshape(NC, NPAD, D)[:, :N_NODES]
    return _tc_pool(s3, y3, dinvb, b3.reshape(1, D), batch3)


# speedup vs baseline: 23.3704x; 1.0183x over previous
import functools

import jax
import jax.numpy as jnp
from jax import lax
from jax.experimental import pallas as pl
from jax.experimental.pallas import tpu as pltpu
from jax.experimental.pallas import tpu_sc as plsc

N_NODES = 10000
N_EDGES = 320000
D = 128
G = 64

NC = 2
NS = 16
NW = NC * NS
EPW = N_EDGES // NW
CHUNK = 80
NCHUNK = EPW // CHUNK
NPAD = 10112
RPT = NPAD // NS
BC = 128
NBC = EPW // BC
BT = EPW - NBC * BC
DEGW = 16

def _sc_scatter_body(y_hbm, src_hbm, dst_hbm, zrows_hbm, out_hbm,
                     s0i, s1i, s2i, d0, d1, d2, sti, dti,
                     rows0, rows1, rows2, acc_sh, sem_g, sem_s):
    cid = lax.axis_index("c")
    sid = lax.axis_index("s")
    wid = sid * NC + cid
    base = wid * EPW

    pltpu.sync_copy(zrows_hbm, rows0)

    def zs(t, _):
        pltpu.sync_copy(rows0, acc_sh.at[pl.ds(sid * RPT + t * BC, BC)])
        return 0

    lax.fori_loop(0, RPT // BC, zs, 0)
    pltpu.sync_copy(rows0.at[pl.ds(0, RPT - (RPT // BC) * BC)],
                    acc_sh.at[pl.ds(sid * RPT + (RPT // BC) * BC,
                                    RPT - (RPT // BC) * BC)])
    plsc.subcore_barrier()

    def load_idx(e, si, di):
        pltpu.sync_copy(src_hbm.at[pl.ds(base + e * BC, BC)], si)
        pltpu.sync_copy(dst_hbm.at[pl.ds(base + e * BC, BC)], di)

    def g_start(si, buf):
        pltpu.async_copy(y_hbm.at[si], buf, sem_g)

    def g_wait(si, buf):
        pltpu.make_async_copy(y_hbm.at[si], buf, sem_g).wait()

    def s_start(buf, d):
        pltpu.async_copy(buf, acc_sh.at[d], sem_s, add=True)

    def s_wait(buf, d):
        pltpu.make_async_copy(buf, acc_sh.at[d], sem_s).wait()

    def step(e, p, q, do_swait=True, do_gstart=True):
        p_rows, p_si, p_d = p
        q_rows, q_si, q_d = q
        if do_swait:
            s_wait(q_rows, q_d)
        if do_gstart:
            load_idx(e + 1, q_si, q_d)
            g_start(q_si, q_rows)
        g_wait(p_si, p_rows)
        s_start(p_rows, p_d)

    sl0 = (rows0, s0i, d0)
    sl1 = (rows1, s1i, d1)
    sl2 = (rows2, s2i, d2)

    load_idx(0, s0i, d0)
    g_start(s0i, rows0)
    step(0, sl0, sl1, do_swait=False)
    step(1, sl1, sl2, do_swait=False)

    def body(i, _):
        e = 3 * i
        step(e + 2, sl2, sl0)
        step(e + 3, sl0, sl1)
        step(e + 4, sl1, sl2)
        return 0

    lax.fori_loop(0, (NBC - 6) // 3, body, 0)

    step(NBC - 4, sl2, sl0)
    step(NBC - 3, sl0, sl1)
    step(NBC - 2, sl1, sl2)
    step(NBC - 1, sl2, sl0, do_gstart=False)

    pltpu.sync_copy(src_hbm.at[pl.ds(base + NBC * BC, BT)], sti)
    pltpu.sync_copy(dst_hbm.at[pl.ds(base + NBC * BC, BT)], dti)
    pltpu.async_copy(y_hbm.at[sti], rows0.at[pl.ds(0, BT)], sem_g)
    pltpu.make_async_copy(y_hbm.at[sti], rows0.at[pl.ds(0, BT)], sem_g).wait()
    pltpu.async_copy(rows0.at[pl.ds(0, BT)], acc_sh.at[dti], sem_s, add=True)
    s_wait(rows1, d1)
    s_wait(rows2, d2)
    pltpu.make_async_copy(rows0.at[pl.ds(0, BT)], acc_sh.at[dti],
                          sem_s).wait()

    plsc.subcore_barrier()

    def ro(t, _):
        r0 = sid * RPT + t * BC
        pltpu.sync_copy(acc_sh.at[pl.ds(r0, BC)], rows0)
        pltpu.sync_copy(rows0, out_hbm.at[pl.ds(cid * NPAD + r0, BC)])
        return 0

    lax.fori_loop(0, RPT // BC, ro, 0)
    rr = RPT - (RPT // BC) * BC
    r0 = sid * RPT + (RPT // BC) * BC
    pltpu.sync_copy(acc_sh.at[pl.ds(r0, rr)], rows0.at[pl.ds(0, rr)])
    pltpu.sync_copy(rows0.at[pl.ds(0, rr)],
                    out_hbm.at[pl.ds(cid * NPAD + r0, rr)])


def _sc_ones_body(ones_hbm, dst_hbm, zdeg_hbm, out_hbm,
                  dA, dB, ones_v, zb_v, acc_sh, sem_s):
    cid = lax.axis_index("c")
    sid = lax.axis_index("s")
    wid = sid * NC + cid
    base = wid * EPW

    pltpu.sync_copy(zdeg_hbm, zb_v)
    pltpu.sync_copy(zb_v, acc_sh.at[pl.ds(sid * RPT, RPT)])
    plsc.subcore_barrier()

    pltpu.sync_copy(ones_hbm, ones_v)

    def load_d(e, d):
        pltpu.sync_copy(dst_hbm.at[pl.ds(base + e * CHUNK, CHUNK)], d)

    def s_start(d):
        pltpu.async_copy(ones_v, acc_sh.at[d], sem_s, add=True)

    def s_wait(d):
        pltpu.make_async_copy(ones_v, acc_sh.at[d], sem_s).wait()

    load_d(0, dA)
    s_start(dA)
    load_d(1, dB)
    s_start(dB)

    def body(i, _):
        s_wait(dA)
        load_d(2 * i, dA)
        s_start(dA)
        s_wait(dB)
        load_d(2 * i + 1, dB)
        s_start(dB)
        return 0

    lax.fori_loop(1, (NCHUNK - 1) // 2, body, 0)

    s_wait(dA)
    load_d(NCHUNK - 1, dA)
    s_start(dA)
    s_wait(dB)
    s_wait(dA)

    plsc.subcore_barrier()
    pltpu.sync_copy(acc_sh.at[pl.ds(sid * RPT, RPT)], zb_v)
    pltpu.sync_copy(zb_v, out_hbm.at[pl.ds(cid * NPAD + sid * RPT, RPT)])


@functools.cache
def _get_sc_scatter():
    mesh = plsc.VectorSubcoreMesh(
        core_axis_name="c", subcore_axis_name="s",
        num_cores=NC, num_subcores=NS,
    )
    return pl.kernel(
        _sc_scatter_body,
        out_type=jax.ShapeDtypeStruct((NC * NPAD, D), jnp.float32),
        mesh=mesh,
        scratch_types=[
            pltpu.VMEM((BC,), jnp.int32),
            pltpu.VMEM((BC,), jnp.int32),
            pltpu.VMEM((BC,), jnp.int32),
            pltpu.VMEM((BC,), jnp.int32),
            pltpu.VMEM((BC,), jnp.int32),
            pltpu.VMEM((BC,), jnp.int32),
            pltpu.VMEM((BT,), jnp.int32),
            pltpu.VMEM((BT,), jnp.int32),
            pltpu.VMEM((BC, D), jnp.float32),
            pltpu.VMEM((BC, D), jnp.float32),
            pltpu.VMEM((BC, D), jnp.float32),
            pltpu.VMEM_SHARED((NPAD, D), jnp.float32),
            pltpu.SemaphoreType.DMA,
            pltpu.SemaphoreType.DMA,
        ],
        name="sc_edge_scatter",
    )


@functools.cache
def _get_sc_ones():
    mesh = plsc.VectorSubcoreMesh(
        core_axis_name="c", subcore_axis_name="s",
        num_cores=NC, num_subcores=NS,
    )
    return pl.kernel(
        _sc_ones_body,
        out_type=jax.ShapeDtypeStruct((NC * NPAD, DEGW), jnp.float32),
        mesh=mesh,
        scratch_types=[
            pltpu.VMEM((CHUNK,), jnp.int32),
            pltpu.VMEM((CHUNK,), jnp.int32),
            pltpu.VMEM((CHUNK, DEGW), jnp.float32),
            pltpu.VMEM((RPT, DEGW), jnp.float32),
            pltpu.VMEM_SHARED((NPAD, DEGW), jnp.float32),
            pltpu.SemaphoreType.DMA,
        ],
        compiler_params=pltpu.CompilerParams(use_tc_tiling_on_sc=False),
        name="sc_degree",
    )


_RB = 2000


def _xw_body(x_ref, w_ref, y_ref):
    y_ref[...] = jnp.dot(
        x_ref[...], w_ref[...], preferred_element_type=jnp.float32
    )


def _tc_xw(x, w):
    return pl.pallas_call(
        _xw_body,
        grid=(N_NODES // _RB,),
        in_specs=[
            pl.BlockSpec((_RB, D), lambda i: (i, 0)),
            pl.BlockSpec((D, D), lambda i: (0, 0)),
        ],
        out_specs=pl.BlockSpec((_RB, D), lambda i: (i, 0)),
        out_shape=jax.ShapeDtypeStruct((N_NODES, D), jnp.float32),
    )(x, w)


def _scale_body(degt_ref, xw_ref, dinvb_ref, y_ref):
    d = degt_ref[0, :, 0:1] + degt_ref[1, :, 0:1] + 1.0
    dinvb = jnp.broadcast_to(lax.rsqrt(d), (_RB, D))
    dinvb_ref[...] = dinvb
    y_ref[...] = xw_ref[...] * dinvb


def _tc_scale(degt, xw):
    return pl.pallas_call(
        _scale_body,
        grid=(N_NODES // _RB,),
        in_specs=[
            pl.BlockSpec((NC, _RB, DEGW), lambda i: (0, i, 0)),
            pl.BlockSpec((_RB, D), lambda i: (i, 0)),
        ],
        out_specs=[
            pl.BlockSpec((_RB, D), lambda i: (i, 0)),
            pl.BlockSpec((_RB, D), lambda i: (i, 0)),
        ],
        out_shape=[
            jax.ShapeDtypeStruct((N_NODES, D), jnp.float32),
            jax.ShapeDtypeStruct((N_NODES, D), jnp.float32),
        ],
    )(degt, xw)


def _mid_body(s_ref, y_ref, dinv_ref, b_ref, w_ref, out_ref):
    h = (s_ref[0] + s_ref[1] + y_ref[...]) * dinv_ref[...] + b_ref[...]
    h = jnp.maximum(h, 0.0)
    out_ref[...] = (
        jnp.dot(h, w_ref[...], preferred_element_type=jnp.float32)
        * dinv_ref[...]
    )


def _tc_mid(s, y, dinvb, b, w):
    return pl.pallas_call(
        _mid_body,
        grid=(N_NODES // _RB,),
        in_specs=[
            pl.BlockSpec((NC, _RB, D), lambda i: (0, i, 0)),
            pl.BlockSpec((_RB, D), lambda i: (i, 0)),
            pl.BlockSpec((_RB, D), lambda i: (i, 0)),
            pl.BlockSpec((1, D), lambda i: (0, 0)),
            pl.BlockSpec((D, D), lambda i: (0, 0)),
        ],
        out_specs=pl.BlockSpec((_RB, D), lambda i: (i, 0)),
        out_shape=jax.ShapeDtypeStruct((N_NODES, D), jnp.float32),
    )(s, y, dinvb, b, w)


_PB = 2000


def _pool_body(s_ref, y_ref, dinv_ref, b_ref, batch_ref, out_ref, sums, counts):
    k = pl.program_id(0)

    @pl.when(k == 0)
    def _():
        sums[...] = jnp.zeros_like(sums)
        counts[...] = jnp.zeros_like(counts)

    h = (s_ref[0] + s_ref[1] + y_ref[...]) * dinv_ref[...] + b_ref[...]
    ids = batch_ref[0, 0, :]
    oh = (ids[None, :] == lax.broadcasted_iota(jnp.int32, (G, _PB), 0)).astype(
        jnp.float32
    )
    sums[...] += jnp.dot(oh, h, preferred_element_type=jnp.float32)
    counts[...] += jnp.broadcast_to(
        jnp.sum(oh, axis=1, keepdims=True), (G, D)
    )
    out_ref[...] = sums[...] / jnp.maximum(counts[...], 1.0)


def _tc_pool(s, y, dinvb, b, batch3):
    return pl.pallas_call(
        _pool_body,
        grid=(N_NODES // _PB,),
        in_specs=[
            pl.BlockSpec((NC, _PB, D), lambda k: (0, k, 0)),
            pl.BlockSpec((_PB, D), lambda k: (k, 0)),
            pl.BlockSpec((_PB, D), lambda k: (k, 0)),
            pl.BlockSpec((1, D), lambda k: (0, 0)),
            pl.BlockSpec((1, 1, _PB), lambda k: (k, 0, 0)),
        ],
        out_specs=pl.BlockSpec((G, D), lambda k: (0, 0)),
        out_shape=jax.ShapeDtypeStruct((G, D), jnp.float32),
        scratch_shapes=[
            pltpu.VMEM((G, D), jnp.float32),
            pltpu.VMEM((G, D), jnp.float32),
        ],
    )(s, y, dinvb, b, batch3)


def kernel(x, edge_index, batch, W1, b1, W2, b2, W3, b3):
    src = edge_index[0].astype(jnp.int32)
    dst = edge_index[1].astype(jnp.int32)
    batch3 = batch.astype(jnp.int32).reshape(N_NODES // _PB, 1, _PB)
    ones_r = jnp.ones((CHUNK, DEGW), jnp.float32)
    zdeg = jnp.zeros((RPT, DEGW), jnp.float32)
    zrows = jnp.zeros((BC, D), jnp.float32)

    sc_scatter = _get_sc_scatter()
    sc_ones = _get_sc_ones()

    xw1 = _tc_xw(x, W1)
    degt = sc_ones(ones_r, dst, zdeg).reshape(NC, NPAD, DEGW)[:, :N_NODES]
    dinvb, y1 = _tc_scale(degt, xw1)
    s1 = sc_scatter(y1, src, dst, zrows).reshape(NC, NPAD, D)[:, :N_NODES]
    y2 = _tc_mid(s1, y1, dinvb, b1.reshape(1, D), W2)
    s2 = sc_scatter(y2, src, dst, zrows).reshape(NC, NPAD, D)[:, :N_NODES]
    y3 = _tc_mid(s2, y2, dinvb, b2.reshape(1, D), W3)
    s3 = sc_scatter(y3, src, dst, zrows).reshape(NC, NPAD, D)[:, :N_NODES]
    return _tc_pool(s3, y3, dinvb, b3.reshape(1, D), batch3)
